# baseline (speedup 1.0000x reference)
import functools

import jax
import jax.numpy as jnp
from jax import lax
from jax.experimental import pallas as pl
from jax.experimental.pallas import tpu as pltpu
from jax.experimental.pallas import tpu_sc as plsc

NBUF = 4
FPAD = 32


@functools.lru_cache(maxsize=None)
def _make_gather(batch: int, fields: int, vocab: int, dim: int):
    info = plsc.get_sparse_core_info()
    num_cores, num_subcores = info.num_cores, info.num_subcores
    num_workers = num_cores * num_subcores
    b_per_w = batch // num_workers
    assert b_per_w * num_workers == batch
    nb = 16
    while b_per_w % nb:
        nb //= 2
    chunk = nb * FPAD
    n_chunks = b_per_w // nb

    mesh = plsc.VectorSubcoreMesh(core_axis_name="c", subcore_axis_name="s")

    @functools.partial(
        pl.kernel,
        out_type=jax.ShapeDtypeStruct((batch, fields, dim), jnp.float32),
        mesh=mesh,
        scratch_types=[
            pltpu.VMEM((b_per_w * FPAD,), jnp.int32),
            pltpu.VMEM((NBUF, chunk, dim), jnp.float32),
            pltpu.SemaphoreType.DMA,
        ]
        + [pltpu.SemaphoreType.DMA for _ in range(2 * NBUF)],
        compiler_params=pltpu.CompilerParams(use_tc_tiling_on_sc=False),
    )
    def gather_kernel(idx_hbm, table_hbm, out_hbm, idx_v, rows_v, sem_i, *sems):
        gsems, psems = sems[:NBUF], sems[NBUF:]
        wid = lax.axis_index("s") * num_cores + lax.axis_index("c")
        base_b = wid * b_per_w
        pltpu.sync_copy(idx_hbm.at[pl.ds(base_b * FPAD, b_per_w * FPAD)], idx_v)

        def start_gather(j):
            buf = j % NBUF
            return pltpu.async_copy(
                table_hbm.at[idx_v.at[pl.ds(j * chunk, chunk)]],
                rows_v.at[buf],
                gsems[buf],
            )

        def start_put(j):
            buf = j % NBUF
            last = None
            for i in range(nb):
                last = pltpu.async_copy(
                    rows_v.at[buf, pl.ds(i * FPAD, fields)],
                    out_hbm.at[base_b + j * nb + i],
                    psems[buf],
                )
            return last

        gathers = [start_gather(j) for j in range(min(NBUF, n_chunks))]
        puts = [None] * n_chunks
        for j in range(n_chunks):
            gathers[j % NBUF].wait()
            puts[j] = start_put(j)
            nxt = j - 1 + NBUF
            if j >= 1 and nxt < n_chunks:
                for _ in range(nb):
                    puts[j - 1].wait()
                gathers[nxt % NBUF] = start_gather(nxt)
        for j in range(max(0, n_chunks - NBUF), n_chunks):
            for _ in range(nb):
                puts[j].wait()

    return gather_kernel


def kernel(inputs, table):
    batch, fields = inputs.shape
    vocab, dim = table.shape
    idx_pad = jnp.pad(inputs.astype(jnp.int32), ((0, 0), (0, FPAD - fields)))
    idx_flat = idx_pad.reshape(-1)
    gather = _make_gather(batch, fields, vocab, dim)
    return gather(idx_flat, table)


# speedup vs baseline: 2.3018x; 2.3018x over previous
import functools

import jax
import jax.numpy as jnp
from jax import lax
from jax.experimental import pallas as pl
from jax.experimental.pallas import tpu as pltpu
from jax.experimental.pallas import tpu_sc as plsc

NBUF = 3
CHUNK = 1024


@functools.lru_cache(maxsize=None)
def _make_gather(n_rows: int, vocab: int, dim: int):
    info = plsc.get_sparse_core_info()
    num_cores, num_subcores = info.num_cores, info.num_subcores
    num_workers = num_cores * num_subcores
    rows_per_worker = n_rows // num_workers
    assert rows_per_worker * num_workers == n_rows
    chunk = min(CHUNK, rows_per_worker)
    n_chunks = rows_per_worker // chunk
    assert n_chunks * chunk == rows_per_worker

    mesh = plsc.VectorSubcoreMesh(core_axis_name="c", subcore_axis_name="s")

    @functools.partial(
        pl.kernel,
        out_type=jax.ShapeDtypeStruct((n_rows, dim), jnp.float32),
        mesh=mesh,
        scratch_types=[
            pltpu.VMEM((rows_per_worker,), jnp.int32),
            pltpu.VMEM((NBUF, chunk, dim), jnp.float32),
            pltpu.SemaphoreType.DMA,
        ]
        + [pltpu.SemaphoreType.DMA for _ in range(2 * NBUF)],
        compiler_params=pltpu.CompilerParams(use_tc_tiling_on_sc=False),
    )
    def gather_kernel(idx_hbm, table_hbm, out_hbm, idx_v, rows_v, sem_i, *sems):
        gsems, psems = sems[:NBUF], sems[NBUF:]
        wid = lax.axis_index("s") * num_cores + lax.axis_index("c")
        base = wid * rows_per_worker
        pltpu.sync_copy(idx_hbm.at[pl.ds(base, rows_per_worker)], idx_v)

        def start_gather(j):
            buf = j % NBUF
            return pltpu.async_copy(
                table_hbm.at[idx_v.at[pl.ds(j * chunk, chunk)]],
                rows_v.at[buf],
                gsems[buf],
            )

        def start_put(j):
            buf = j % NBUF
            return pltpu.async_copy(
                rows_v.at[buf],
                out_hbm.at[pl.ds(base + j * chunk, chunk)],
                psems[buf],
            )

        gathers = [start_gather(j) for j in range(min(NBUF, n_chunks))]
        puts = [None] * n_chunks
        for j in range(n_chunks):
            gathers[j % NBUF].wait()
            puts[j] = start_put(j)
            nxt = j - 1 + NBUF
            if j >= 1 and nxt < n_chunks:
                puts[j - 1].wait()
                gathers[nxt % NBUF] = start_gather(nxt)
        for j in range(max(0, n_chunks - NBUF), n_chunks):
            puts[j].wait()

    return gather_kernel


def kernel(inputs, table):
    batch, fields = inputs.shape
    vocab, dim = table.shape
    idx_flat = inputs.reshape(-1).astype(jnp.int32)
    gather = _make_gather(idx_flat.shape[0], vocab, dim)
    out = gather(idx_flat, table)
    return out.reshape(batch, fields, dim)
